# baseline (speedup 1.0000x reference)
import dataclasses
import functools

import jax
import jax.numpy as jnp
from jax import lax
from jax.experimental import pallas as pl
from jax.experimental.pallas import tpu as pltpu
from jax.experimental.pallas import tpu_sc as plsc

NC = 2
NS = 16
NW = NC * NS
BLK = 128


def _deg_kernel(np_, nblk):
  hrows = np_ // BLK
  rps = 8
  nw_io = hrows // rps
  mesh = plsc.VectorSubcoreMesh(core_axis_name="c", subcore_axis_name="s")
  cp = pltpu.CompilerParams()
  if "needs_layout_passes" in pltpu.CompilerParams.__dataclass_fields__:
    cp = dataclasses.replace(cp, needs_layout_passes=False)

  def body(dst_hbm, iota_hbm, zeros_hbm, out_hbm, idx_d, hist_v, idx80_v,
           acc_sh):
    cid = lax.axis_index("c")
    sid = lax.axis_index("s")
    wid = cid * NS + sid
    pltpu.sync_copy(dst_hbm.at[pl.ds(wid * nblk, nblk)], idx_d)
    pltpu.sync_copy(zeros_hbm, hist_v)
    pltpu.sync_copy(iota_hbm, idx80_v)

    @pl.when(sid < nw_io)
    def _():
      pltpu.sync_copy(zeros_hbm.at[pl.ds(0, rps)],
                      acc_sh.at[pl.ds(sid * rps, rps)])

    plsc.subcore_barrier()

    ones16 = jnp.ones((16,), jnp.float32)

    @pl.loop(0, nblk)
    def _(j):
      for k in range(BLK // 16):
        dvec = idx_d[j, pl.ds(k * 16, 16)]
        plsc.addupdate_scatter(
            hist_v, [jnp.right_shift(dvec, 7), jnp.bitwise_and(dvec, 127)],
            ones16)

    pltpu.sync_copy(hist_v, acc_sh.at[idx80_v], add=True)
    plsc.subcore_barrier()

    @pl.when(sid < nw_io)
    def _():
      pltpu.sync_copy(acc_sh.at[pl.ds(sid * rps, rps)],
                      out_hbm.at[pl.ds(cid * hrows + sid * rps, rps)])

  return pl.kernel(
      body, mesh=mesh, compiler_params=cp,
      out_type=jax.ShapeDtypeStruct((NC * hrows, BLK), jnp.float32),
      scratch_types=[
          pltpu.VMEM((nblk, BLK), jnp.int32),
          pltpu.VMEM((hrows, BLK), jnp.float32),
          pltpu.VMEM((hrows,), jnp.int32),
          pltpu.VMEM_SHARED((hrows, BLK), jnp.float32),
      ])


def _edge_scatter_kernel(np_, d, nblk):
  rps = np_ // NS
  mesh = plsc.VectorSubcoreMesh(core_axis_name="c", subcore_axis_name="s")

  def body(g_hbm, src_hbm, dst_hbm, zeros_hbm, out_hbm, idx_s, idx_d,
           rows0, rows1, sem0, sem1, acc_sh):
    cid = lax.axis_index("c")
    sid = lax.axis_index("s")
    wid = cid * NS + sid
    pltpu.sync_copy(src_hbm.at[pl.ds(wid * nblk, nblk)], idx_s)
    pltpu.sync_copy(dst_hbm.at[pl.ds(wid * nblk, nblk)], idx_d)
    pltpu.async_copy(g_hbm.at[idx_s.at[0]], rows0, sem0)
    pltpu.async_copy(g_hbm.at[idx_s.at[1]], rows1, sem1)
    pltpu.sync_copy(zeros_hbm, acc_sh.at[pl.ds(sid * rps, rps)])
    plsc.subcore_barrier()

    @pl.loop(0, nblk // 2 - 1)
    def _(jj):
      j0 = 2 * jj
      pltpu.make_async_copy(g_hbm.at[idx_s.at[j0]], rows0, sem0).wait()
      pltpu.sync_copy(rows0, acc_sh.at[idx_d.at[j0]], add=True)
      pltpu.async_copy(g_hbm.at[idx_s.at[j0 + 2]], rows0, sem0)
      pltpu.make_async_copy(g_hbm.at[idx_s.at[j0 + 1]], rows1, sem1).wait()
      pltpu.sync_copy(rows1, acc_sh.at[idx_d.at[j0 + 1]], add=True)
      pltpu.async_copy(g_hbm.at[idx_s.at[j0 + 3]], rows1, sem1)

    pltpu.make_async_copy(g_hbm.at[idx_s.at[nblk - 2]], rows0, sem0).wait()
    pltpu.sync_copy(rows0, acc_sh.at[idx_d.at[nblk - 2]], add=True)
    pltpu.make_async_copy(g_hbm.at[idx_s.at[nblk - 1]], rows1, sem1).wait()
    pltpu.sync_copy(rows1, acc_sh.at[idx_d.at[nblk - 1]], add=True)

    plsc.subcore_barrier()
    pltpu.sync_copy(acc_sh.at[pl.ds(sid * rps, rps)],
                    out_hbm.at[pl.ds(cid * np_ + sid * rps, rps)])

  return pl.kernel(
      body, mesh=mesh,
      out_type=jax.ShapeDtypeStruct((NC * np_, d), jnp.float32),
      scratch_types=[
          pltpu.VMEM((nblk, BLK), jnp.int32),
          pltpu.VMEM((nblk, BLK), jnp.int32),
          pltpu.VMEM((BLK, d), jnp.float32),
          pltpu.VMEM((BLK, d), jnp.float32),
          pltpu.SemaphoreType.DMA,
          pltpu.SemaphoreType.DMA,
          pltpu.VMEM_SHARED((np_, d), jnp.float32),
      ])


def _b_body(d0_ref, d1_ref, x_ref, g0_ref, dinv_ref):
  deg = d0_ref[...] + d1_ref[...] + 1.0
  dinv = lax.rsqrt(deg)
  dinv_ref[...] = dinv
  g0_ref[...] = jnp.broadcast_to(dinv, x_ref.shape) * x_ref[...]


def _d_body(s0a_ref, s0b_ref, x_ref, dinv_ref, w1_ref, b1_ref, w2_ref,
            g1_ref, u_ref):
  dinv = jnp.broadcast_to(dinv_ref[...], x_ref.shape)
  y0 = dinv * (s0a_ref[...] + s0b_ref[...]) + dinv * dinv * x_ref[...]
  h = jnp.dot(y0, w1_ref[...], preferred_element_type=jnp.float32,
              precision=lax.Precision.HIGHEST) + b1_ref[...]
  z = jnp.maximum(h, 0.0)
  u = jnp.dot(z, w2_ref[...], preferred_element_type=jnp.float32,
              precision=lax.Precision.HIGHEST)
  u_ref[...] = u
  g1_ref[...] = dinv * u


def _f_body(s1a_ref, s1b_ref, u_ref, dinv_ref, b2_ref, out_ref):
  dinv = jnp.broadcast_to(dinv_ref[...], u_ref.shape)
  v = dinv * (s1a_ref[...] + s1b_ref[...]) + dinv * dinv * u_ref[...]
  v = v + b2_ref[...]
  d = v.shape[1]
  dd = d // 4
  outs = []
  for k in range(4):
    vk = v[:, k * dd:(k + 1) * dd]
    n2 = jnp.sum(vk * vk, axis=1, keepdims=True)
    nr = jnp.maximum(jnp.sqrt(n2), 1e-12)
    outs.append(vk / nr)
  out_ref[...] = jnp.concatenate(outs, axis=1)


def kernel(edge_index, x_list, ix, aug_loss, W1, b1, W2, b2):
  n, d = x_list.shape
  e = edge_index.shape[1]
  d4 = W1.shape[1]

  np_ = ((n + 1279) // 1280) * 1280
  epw = ((e + NW * 2 * BLK - 1) // (NW * 2 * BLK)) * 2 * BLK
  ep = epw * NW
  nblk = epw // BLK

  pad_idx = (n + jnp.arange(ep - e, dtype=jnp.int32) % (np_ - n)).astype(
      jnp.int32)
  src = jnp.concatenate([edge_index[0], pad_idx]).reshape(ep // BLK, BLK)
  dst = jnp.concatenate([edge_index[1], pad_idx]).reshape(ep // BLK, BLK)
  xp = jnp.pad(x_list, ((0, np_ - n), (0, 0)))
  zerosd = jnp.zeros((np_ // NS, d), jnp.float32)
  hrows = np_ // BLK
  iota80 = jnp.arange(hrows, dtype=jnp.int32)
  zerosh = jnp.zeros((hrows, BLK), jnp.float32)
  b1r = b1.reshape(1, d4)
  b2r = b2.reshape(1, d)

  degp = _deg_kernel(np_, nblk)(dst, iota80, zerosh)
  d0 = degp[:hrows].reshape(np_, 1)
  d1 = degp[hrows:].reshape(np_, 1)

  rb = 1024
  grid = (np_ // rb,)
  g0, dinvb = pl.pallas_call(
      _b_body,
      grid=grid,
      in_specs=[
          pl.BlockSpec((rb, 1), lambda i: (i, 0)),
          pl.BlockSpec((rb, 1), lambda i: (i, 0)),
          pl.BlockSpec((rb, d), lambda i: (i, 0)),
      ],
      out_specs=[
          pl.BlockSpec((rb, d), lambda i: (i, 0)),
          pl.BlockSpec((rb, 1), lambda i: (i, 0)),
      ],
      out_shape=[
          jax.ShapeDtypeStruct((np_, d), jnp.float32),
          jax.ShapeDtypeStruct((np_, 1), jnp.float32),
      ],
  )(d0, d1, xp)

  scat = _edge_scatter_kernel(np_, d, (ep // BLK) // NW)
  s0 = scat(g0, src, dst, zerosd)

  hb = np_ // rb
  g1, u = pl.pallas_call(
      _d_body,
      grid=grid,
      in_specs=[
          pl.BlockSpec((rb, d), lambda i: (i, 0)),
          pl.BlockSpec((rb, d), lambda i: (i + hb, 0)),
          pl.BlockSpec((rb, d), lambda i: (i, 0)),
          pl.BlockSpec((rb, 1), lambda i: (i, 0)),
          pl.BlockSpec((d, d4), lambda i: (0, 0)),
          pl.BlockSpec((1, d4), lambda i: (0, 0)),
          pl.BlockSpec((d4, d), lambda i: (0, 0)),
      ],
      out_specs=[
          pl.BlockSpec((rb, d), lambda i: (i, 0)),
          pl.BlockSpec((rb, d), lambda i: (i, 0)),
      ],
      out_shape=[
          jax.ShapeDtypeStruct((np_, d), jnp.float32),
          jax.ShapeDtypeStruct((np_, d), jnp.float32),
      ],
  )(s0, s0, xp, dinvb, W1, b1r, W2)

  s1 = scat(g1, src, dst, zerosd)

  out = pl.pallas_call(
      _f_body,
      grid=grid,
      in_specs=[
          pl.BlockSpec((rb, d), lambda i: (i, 0)),
          pl.BlockSpec((rb, d), lambda i: (i + hb, 0)),
          pl.BlockSpec((rb, d), lambda i: (i, 0)),
          pl.BlockSpec((rb, 1), lambda i: (i, 0)),
          pl.BlockSpec((1, d), lambda i: (0, 0)),
      ],
      out_specs=pl.BlockSpec((rb, d), lambda i: (i, 0)),
      out_shape=jax.ShapeDtypeStruct((np_, d), jnp.float32),
  )(s1, s1, u, dinvb, b2r)

  return out[:n]


# speedup vs baseline: 1.1567x; 1.1567x over previous
import dataclasses
import functools

import jax
import jax.numpy as jnp
from jax import lax
from jax.experimental import pallas as pl
from jax.experimental.pallas import tpu as pltpu
from jax.experimental.pallas import tpu_sc as plsc

NC = 2
NS = 16
NW = NC * NS
BLK = 128


def _deg_kernel(np_, nblk):
  hrows = np_ // BLK
  rps = 8
  nw_io = hrows // rps
  mesh = plsc.VectorSubcoreMesh(core_axis_name="c", subcore_axis_name="s")
  cp = pltpu.CompilerParams()
  if "needs_layout_passes" in pltpu.CompilerParams.__dataclass_fields__:
    cp = dataclasses.replace(cp, needs_layout_passes=False)

  def body(dst_hbm, iota_hbm, zeros_hbm, out_hbm, idx_d, hist_v, idx80_v,
           acc_sh):
    cid = lax.axis_index("c")
    sid = lax.axis_index("s")
    wid = cid * NS + sid
    pltpu.sync_copy(dst_hbm.at[pl.ds(wid * nblk, nblk)], idx_d)
    pltpu.sync_copy(zeros_hbm, hist_v)
    pltpu.sync_copy(iota_hbm, idx80_v)

    @pl.when(sid < nw_io)
    def _():
      pltpu.sync_copy(zeros_hbm.at[pl.ds(0, rps)],
                      acc_sh.at[pl.ds(sid * rps, rps)])

    plsc.subcore_barrier()

    ones16 = jnp.ones((16,), jnp.float32)

    @pl.loop(0, nblk)
    def _(j):
      for k in range(BLK // 16):
        dvec = idx_d[j, pl.ds(k * 16, 16)]
        plsc.addupdate_scatter(
            hist_v, [jnp.right_shift(dvec, 7), jnp.bitwise_and(dvec, 127)],
            ones16)

    pltpu.sync_copy(hist_v, acc_sh.at[idx80_v], add=True)
    plsc.subcore_barrier()

    @pl.when(sid < nw_io)
    def _():
      pltpu.sync_copy(acc_sh.at[pl.ds(sid * rps, rps)],
                      out_hbm.at[pl.ds(cid * hrows + sid * rps, rps)])

  return pl.kernel(
      body, mesh=mesh, compiler_params=cp,
      out_type=jax.ShapeDtypeStruct((NC * hrows, BLK), jnp.float32),
      scratch_types=[
          pltpu.VMEM((nblk, BLK), jnp.int32),
          pltpu.VMEM((hrows, BLK), jnp.float32),
          pltpu.VMEM((hrows,), jnp.int32),
          pltpu.VMEM_SHARED((hrows, BLK), jnp.float32),
      ])


def _edge_scatter_kernel(np_, d, nblk):
  rps = np_ // NS
  mesh = plsc.VectorSubcoreMesh(core_axis_name="c", subcore_axis_name="s")

  def body(g_hbm, src_hbm, dst_hbm, zeros_hbm, out_hbm, idx_s, idx_d,
           rows0, rows1, sem0, sem1, acc_sh):
    cid = lax.axis_index("c")
    sid = lax.axis_index("s")
    wid = cid * NS + sid
    pltpu.sync_copy(src_hbm.at[pl.ds(wid * nblk, nblk)], idx_s)
    pltpu.sync_copy(dst_hbm.at[pl.ds(wid * nblk, nblk)], idx_d)
    pltpu.async_copy(g_hbm.at[idx_s.at[0]], rows0, sem0)
    pltpu.async_copy(g_hbm.at[idx_s.at[1]], rows1, sem1)
    pltpu.sync_copy(zeros_hbm, acc_sh.at[pl.ds(sid * rps, rps)])
    plsc.subcore_barrier()

    @pl.loop(0, nblk // 2 - 1)
    def _(jj):
      j0 = 2 * jj
      pltpu.make_async_copy(g_hbm.at[idx_s.at[j0]], rows0, sem0).wait()
      pltpu.sync_copy(rows0, acc_sh.at[idx_d.at[j0]], add=True)
      pltpu.async_copy(g_hbm.at[idx_s.at[j0 + 2]], rows0, sem0)
      pltpu.make_async_copy(g_hbm.at[idx_s.at[j0 + 1]], rows1, sem1).wait()
      pltpu.sync_copy(rows1, acc_sh.at[idx_d.at[j0 + 1]], add=True)
      pltpu.async_copy(g_hbm.at[idx_s.at[j0 + 3]], rows1, sem1)

    pltpu.make_async_copy(g_hbm.at[idx_s.at[nblk - 2]], rows0, sem0).wait()
    pltpu.sync_copy(rows0, acc_sh.at[idx_d.at[nblk - 2]], add=True)
    pltpu.make_async_copy(g_hbm.at[idx_s.at[nblk - 1]], rows1, sem1).wait()
    pltpu.sync_copy(rows1, acc_sh.at[idx_d.at[nblk - 1]], add=True)

    plsc.subcore_barrier()
    pltpu.sync_copy(acc_sh.at[pl.ds(sid * rps, rps)],
                    out_hbm.at[pl.ds(cid * np_ + sid * rps, rps)])

  return pl.kernel(
      body, mesh=mesh,
      out_type=jax.ShapeDtypeStruct((NC * np_, d), jnp.float32),
      scratch_types=[
          pltpu.VMEM((nblk, BLK), jnp.int32),
          pltpu.VMEM((nblk, BLK), jnp.int32),
          pltpu.VMEM((BLK, d), jnp.float32),
          pltpu.VMEM((BLK, d), jnp.float32),
          pltpu.SemaphoreType.DMA,
          pltpu.SemaphoreType.DMA,
          pltpu.VMEM_SHARED((np_, d), jnp.float32),
      ])


def _b_body(d0_ref, d1_ref, x_ref, g0_ref, dinv_ref):
  deg = d0_ref[...] + d1_ref[...] + 1.0
  dinv = lax.rsqrt(deg)
  dinv_ref[...] = dinv
  g0_ref[...] = jnp.broadcast_to(dinv, x_ref.shape) * x_ref[...]


def _d_body(s0a_ref, s0b_ref, x_ref, dinv_ref, w1_ref, b1_ref, w2_ref,
            g1_ref, u_ref):
  dinv = jnp.broadcast_to(dinv_ref[...], x_ref.shape)
  y0 = dinv * (s0a_ref[...] + s0b_ref[...]) + dinv * dinv * x_ref[...]
  h = jnp.dot(y0, w1_ref[...], preferred_element_type=jnp.float32) + b1_ref[...]
  z = jnp.maximum(h, 0.0)
  u = jnp.dot(z, w2_ref[...], preferred_element_type=jnp.float32)
  u_ref[...] = u
  g1_ref[...] = dinv * u


def _f_body(s1a_ref, s1b_ref, u_ref, dinv_ref, b2_ref, out_ref):
  dinv = jnp.broadcast_to(dinv_ref[...], u_ref.shape)
  v = dinv * (s1a_ref[...] + s1b_ref[...]) + dinv * dinv * u_ref[...]
  v = v + b2_ref[...]
  d = v.shape[1]
  dd = d // 4
  outs = []
  for k in range(4):
    vk = v[:, k * dd:(k + 1) * dd]
    n2 = jnp.sum(vk * vk, axis=1, keepdims=True)
    nr = jnp.maximum(jnp.sqrt(n2), 1e-12)
    outs.append(vk / nr)
  out_ref[...] = jnp.concatenate(outs, axis=1)


def kernel(edge_index, x_list, ix, aug_loss, W1, b1, W2, b2):
  n, d = x_list.shape
  e = edge_index.shape[1]
  d4 = W1.shape[1]

  np_ = ((n + 1279) // 1280) * 1280
  epw = ((e + NW * 2 * BLK - 1) // (NW * 2 * BLK)) * 2 * BLK
  ep = epw * NW
  nblk = epw // BLK

  pad_idx = (n + jnp.arange(ep - e, dtype=jnp.int32) % (np_ - n)).astype(
      jnp.int32)
  src = jnp.concatenate([edge_index[0], pad_idx]).reshape(ep // BLK, BLK)
  dst = jnp.concatenate([edge_index[1], pad_idx]).reshape(ep // BLK, BLK)
  xp = jnp.pad(x_list, ((0, np_ - n), (0, 0)))
  zerosd = jnp.zeros((np_ // NS, d), jnp.float32)
  hrows = np_ // BLK
  iota80 = jnp.arange(hrows, dtype=jnp.int32)
  zerosh = jnp.zeros((hrows, BLK), jnp.float32)
  b1r = b1.reshape(1, d4)
  b2r = b2.reshape(1, d)

  degp = _deg_kernel(np_, nblk)(dst, iota80, zerosh)
  d0 = degp[:hrows].reshape(np_, 1)
  d1 = degp[hrows:].reshape(np_, 1)

  rb = 1024
  grid = (np_ // rb,)
  g0, dinvb = pl.pallas_call(
      _b_body,
      grid=grid,
      in_specs=[
          pl.BlockSpec((rb, 1), lambda i: (i, 0)),
          pl.BlockSpec((rb, 1), lambda i: (i, 0)),
          pl.BlockSpec((rb, d), lambda i: (i, 0)),
      ],
      out_specs=[
          pl.BlockSpec((rb, d), lambda i: (i, 0)),
          pl.BlockSpec((rb, 1), lambda i: (i, 0)),
      ],
      out_shape=[
          jax.ShapeDtypeStruct((np_, d), jnp.float32),
          jax.ShapeDtypeStruct((np_, 1), jnp.float32),
      ],
  )(d0, d1, xp)

  scat = _edge_scatter_kernel(np_, d, (ep // BLK) // NW)
  s0 = scat(g0, src, dst, zerosd)

  hb = np_ // rb
  g1, u = pl.pallas_call(
      _d_body,
      grid=grid,
      in_specs=[
          pl.BlockSpec((rb, d), lambda i: (i, 0)),
          pl.BlockSpec((rb, d), lambda i: (i + hb, 0)),
          pl.BlockSpec((rb, d), lambda i: (i, 0)),
          pl.BlockSpec((rb, 1), lambda i: (i, 0)),
          pl.BlockSpec((d, d4), lambda i: (0, 0)),
          pl.BlockSpec((1, d4), lambda i: (0, 0)),
          pl.BlockSpec((d4, d), lambda i: (0, 0)),
      ],
      out_specs=[
          pl.BlockSpec((rb, d), lambda i: (i, 0)),
          pl.BlockSpec((rb, d), lambda i: (i, 0)),
      ],
      out_shape=[
          jax.ShapeDtypeStruct((np_, d), jnp.float32),
          jax.ShapeDtypeStruct((np_, d), jnp.float32),
      ],
  )(s0, s0, xp, dinvb, W1, b1r, W2)

  s1 = scat(g1, src, dst, zerosd)

  out = pl.pallas_call(
      _f_body,
      grid=grid,
      in_specs=[
          pl.BlockSpec((rb, d), lambda i: (i, 0)),
          pl.BlockSpec((rb, d), lambda i: (i + hb, 0)),
          pl.BlockSpec((rb, d), lambda i: (i, 0)),
          pl.BlockSpec((rb, 1), lambda i: (i, 0)),
          pl.BlockSpec((1, d), lambda i: (0, 0)),
      ],
      out_specs=pl.BlockSpec((rb, d), lambda i: (i, 0)),
      out_shape=jax.ShapeDtypeStruct((np_, d), jnp.float32),
  )(s1, s1, u, dinvb, b2r)

  return out[:n]
